# baseline (speedup 1.0000x reference)
import functools

import numpy as np

import jax
import jax.numpy as jnp
from jax.experimental import pallas as pl
from jax.experimental.pallas import tpu as pltpu

_SOBEL_EPS = 1e-6

_LANE_L1_FINE = 0
_LANE_SOB_FINE = 1
_LANE_L1_COARSE = 2
_LANE_SOB_COARSE = 3
_LANE_ERR = 4
_OUT_SUBLANES = 8
_OUT_LANES = 128


def _shift_down(x):
    return jnp.concatenate([x[:1], x[:-1]], axis=0)


def _shift_up(x):
    return jnp.concatenate([x[1:], x[-1:]], axis=0)


def _shift_right(x):
    return jnp.concatenate([x[:, :1], x[:, :-1]], axis=1)


def _shift_left(x):
    return jnp.concatenate([x[:, 1:], x[:, -1:]], axis=1)


def _sobel_mag8(x):
    x_u = _shift_down(x)
    x_d = _shift_up(x)
    sv = x_u + 2.0 * x + x_d
    dv = x_d - x_u
    gx = _shift_left(sv) - _shift_right(sv)
    gy = _shift_right(dv) + 2.0 * dv + _shift_left(dv)
    g2 = gx * gx + gy * gy + 64.0 * _SOBEL_EPS
    return g2 * jax.lax.rsqrt(g2)


def _loss_body(p_ref, t_ref, c_ref, e_ref, mh_ref, mw_ref, out_ref,
               *, img_h, low_h, n_img):
    mh = mh_ref[...]
    mw = mw_ref[...]

    def upsample(ref, i):
        xb = ref[i * low_h:(i + 1) * low_h].astype(jnp.bfloat16)
        y = jnp.dot(xb, mw, preferred_element_type=jnp.float32)
        return jnp.dot(mh, y.astype(jnp.bfloat16),
                       preferred_element_type=jnp.float32).astype(jnp.bfloat16)

    sums = [jnp.float32(0.0)] * 5

    def fsum(x):
        h, w = x.shape
        x = x[:h // 2] + x[h // 2:]
        x = x[:, :w // 2] + x[:, w // 2:]
        return jnp.sum(x, dtype=jnp.float32)
    for i in range(n_img):
        sl = slice(i * img_h, (i + 1) * img_h)
        p = p_ref[sl].astype(jnp.bfloat16)
        t = t_ref[sl].astype(jnp.bfloat16)
        c = upsample(c_ref, i)
        e = upsample(e_ref, i)

        mag_t = _sobel_mag8(t)
        mag_p = _sobel_mag8(p)
        mag_c = _sobel_mag8(c)

        ct = jnp.abs(c - t)
        sums[0] += fsum(jnp.abs(p - t))
        sums[1] += 0.125 * fsum(jnp.abs(mag_p - mag_t))
        sums[2] += fsum(ct)
        sums[3] += 0.125 * fsum(jnp.abs(mag_c - mag_t))
        sums[4] += fsum(jnp.abs(e - ct))

    lane = jax.lax.broadcasted_iota(
        jnp.int32, (_OUT_SUBLANES, _OUT_LANES), 1)
    acc = jnp.zeros((_OUT_SUBLANES, _OUT_LANES), jnp.float32)
    for idx, s in zip((_LANE_L1_FINE, _LANE_SOB_FINE, _LANE_L1_COARSE,
                       _LANE_SOB_COARSE, _LANE_ERR), sums):
        acc = acc + jnp.where(lane == idx, s, 0.0)

    @pl.when(pl.program_id(0) == 0)
    def _init():
        out_ref[...] = acc

    @pl.when(pl.program_id(0) != 0)
    def _accum():
        out_ref[...] += acc


def _bilinear_operator(n_src, n_dst):
    i = np.arange(n_dst, dtype=np.float64)
    src = (i + 0.5) * (n_src / n_dst) - 0.5
    lo = np.floor(src).astype(np.int64)
    w = src - lo
    hi = np.clip(lo + 1, 0, n_src - 1)
    lo = np.clip(lo, 0, n_src - 1)
    m = np.zeros((n_dst, n_src), dtype=np.float32)
    m[i.astype(np.int64), lo] += (1.0 - w).astype(np.float32)
    m[i.astype(np.int64), hi] += w.astype(np.float32)
    return m


def kernel(pred_pha_fine, true_pha, pred_pha_corse, pred_err):
    B, C, H, W = true_pha.shape
    bc = B * C
    hh, ww = pred_pha_corse.shape[2:]
    tbc = next(d for d in (16, 8, 4, 2, 1) if bc % d == 0)
    nblk = bc // tbc

    p2 = pred_pha_fine.reshape(bc * H, W)
    t2 = true_pha.reshape(bc * H, W)
    c2 = pred_pha_corse.reshape(bc * hh, ww)
    e2 = pred_err.reshape(bc * hh, ww)
    mh = jnp.asarray(_bilinear_operator(hh, H), dtype=jnp.bfloat16)
    mw = jnp.asarray(_bilinear_operator(ww, W).T, dtype=jnp.bfloat16)

    full_spec = pl.BlockSpec((tbc * H, W), lambda i: (i, 0))
    half_spec = pl.BlockSpec((tbc * hh, ww), lambda i: (i, 0))
    fixed = lambda shape: pl.BlockSpec(shape, lambda i: (0, 0))

    out_spec = pl.BlockSpec((_OUT_SUBLANES, _OUT_LANES), lambda i: (0, 0))
    out_shape = jax.ShapeDtypeStruct((_OUT_SUBLANES, _OUT_LANES), jnp.float32)

    body = functools.partial(_loss_body, img_h=H, low_h=hh, n_img=tbc)
    sums = pl.pallas_call(
        body,
        out_shape=out_shape,
        grid=(nblk,),
        in_specs=[full_spec, full_spec, half_spec, half_spec,
                  fixed(mh.shape), fixed(mw.shape)],
        out_specs=out_spec,
        compiler_params=pltpu.CompilerParams(
            dimension_semantics=("arbitrary",),
            vmem_limit_bytes=96 * 1024 * 1024),
    )(p2, t2, c2, e2, mh, mw)

    n = jnp.float32(bc * H * W)
    row = sums[0]
    return {
        'main_loss': (row[_LANE_L1_FINE] + row[_LANE_SOB_FINE]) / n,
        'coarse_pred_loss': (row[_LANE_L1_COARSE] + row[_LANE_SOB_COARSE]) / n,
        'pred_err_loss': row[_LANE_ERR] / n,
    }


# speedup vs baseline: 1.0133x; 1.0133x over previous
import functools

import numpy as np

import jax
import jax.numpy as jnp
from jax.experimental import pallas as pl
from jax.experimental.pallas import tpu as pltpu

_SOBEL_EPS = 1e-6

_LANE_L1_FINE = 0
_LANE_SOB_FINE = 1
_LANE_L1_COARSE = 2
_LANE_SOB_COARSE = 3
_LANE_ERR = 4
_OUT_SUBLANES = 8
_OUT_LANES = 128


def _shift_down(x):
    return jnp.concatenate([x[:1], x[:-1]], axis=0)


def _shift_up(x):
    return jnp.concatenate([x[1:], x[-1:]], axis=0)


def _shift_right(x):
    return jnp.concatenate([x[:, :1], x[:, :-1]], axis=1)


def _shift_left(x):
    return jnp.concatenate([x[:, 1:], x[:, -1:]], axis=1)


def _sobel_mag8(x):
    x_u = _shift_down(x)
    x_d = _shift_up(x)
    sv = x_u + 2.0 * x + x_d
    dv = x_d - x_u
    gx = _shift_left(sv) - _shift_right(sv)
    gy = _shift_right(dv) + 2.0 * dv + _shift_left(dv)
    g2 = gx * gx + gy * gy + 64.0 * _SOBEL_EPS
    return g2 * jax.lax.rsqrt(g2)


def _loss_body(p_ref, t_ref, c_ref, e_ref, mh_ref, mw_ref, out_ref,
               *, img_h, low_h, n_img):
    mh = mh_ref[...]
    mw = mw_ref[...]

    def upsample(ref, i):
        xb = ref[i * low_h:(i + 1) * low_h].astype(jnp.bfloat16)
        y = jnp.dot(xb, mw, preferred_element_type=jnp.float32)
        return jnp.dot(mh, y.astype(jnp.bfloat16),
                       preferred_element_type=jnp.float32).astype(jnp.bfloat16)

    sums = [jnp.float32(0.0)] * 5

    def fsum(x):
        h, w = x.shape
        x = x[:h // 2] + x[h // 2:]
        x = x[:, :w // 2] + x[:, w // 2:]
        return jnp.sum(x, dtype=jnp.float32)
    for i in range(n_img):
        sl = slice(i * img_h, (i + 1) * img_h)
        p = p_ref[sl].astype(jnp.bfloat16)
        t = t_ref[sl].astype(jnp.bfloat16)
        c = upsample(c_ref, i)
        e = upsample(e_ref, i)

        mag_t = _sobel_mag8(t)
        mag_p = _sobel_mag8(p)
        mag_c = _sobel_mag8(c)

        ct = jnp.abs(c - t)
        sums[0] += fsum(jnp.abs(p - t))
        sums[1] += 0.125 * fsum(jnp.abs(mag_p - mag_t))
        sums[2] += fsum(ct)
        sums[3] += 0.125 * fsum(jnp.abs(mag_c - mag_t))
        sums[4] += fsum(jnp.abs(e - ct))

    lane = jax.lax.broadcasted_iota(
        jnp.int32, (_OUT_SUBLANES, _OUT_LANES), 1)
    acc = jnp.zeros((_OUT_SUBLANES, _OUT_LANES), jnp.float32)
    for idx, s in zip((_LANE_L1_FINE, _LANE_SOB_FINE, _LANE_L1_COARSE,
                       _LANE_SOB_COARSE, _LANE_ERR), sums):
        acc = acc + jnp.where(lane == idx, s, 0.0)

    @pl.when(pl.program_id(0) == 0)
    def _init():
        out_ref[...] = acc

    @pl.when(pl.program_id(0) != 0)
    def _accum():
        out_ref[...] += acc


def _bilinear_operator(n_src, n_dst):
    i = np.arange(n_dst, dtype=np.float64)
    src = (i + 0.5) * (n_src / n_dst) - 0.5
    lo = np.floor(src).astype(np.int64)
    w = src - lo
    hi = np.clip(lo + 1, 0, n_src - 1)
    lo = np.clip(lo, 0, n_src - 1)
    m = np.zeros((n_dst, n_src), dtype=np.float32)
    m[i.astype(np.int64), lo] += (1.0 - w).astype(np.float32)
    m[i.astype(np.int64), hi] += w.astype(np.float32)
    return m


def kernel(pred_pha_fine, true_pha, pred_pha_corse, pred_err):
    B, C, H, W = true_pha.shape
    bc = B * C
    hh, ww = pred_pha_corse.shape[2:]
    tbc = next(d for d in (8, 4, 2, 1) if bc % d == 0)
    nblk = bc // tbc

    p2 = pred_pha_fine.reshape(bc * H, W)
    t2 = true_pha.reshape(bc * H, W)
    c2 = pred_pha_corse.reshape(bc * hh, ww)
    e2 = pred_err.reshape(bc * hh, ww)
    mh = jnp.asarray(_bilinear_operator(hh, H), dtype=jnp.bfloat16)
    mw = jnp.asarray(_bilinear_operator(ww, W).T, dtype=jnp.bfloat16)

    full_spec = pl.BlockSpec((tbc * H, W), lambda i: (i, 0))
    half_spec = pl.BlockSpec((tbc * hh, ww), lambda i: (i, 0))
    fixed = lambda shape: pl.BlockSpec(shape, lambda i: (0, 0))

    out_spec = pl.BlockSpec((_OUT_SUBLANES, _OUT_LANES), lambda i: (0, 0))
    out_shape = jax.ShapeDtypeStruct((_OUT_SUBLANES, _OUT_LANES), jnp.float32)

    body = functools.partial(_loss_body, img_h=H, low_h=hh, n_img=tbc)
    sums = pl.pallas_call(
        body,
        out_shape=out_shape,
        grid=(nblk,),
        in_specs=[full_spec, full_spec, half_spec, half_spec,
                  fixed(mh.shape), fixed(mw.shape)],
        out_specs=out_spec,
        compiler_params=pltpu.CompilerParams(
            dimension_semantics=("arbitrary",),
            vmem_limit_bytes=96 * 1024 * 1024),
    )(p2, t2, c2, e2, mh, mw)

    n = jnp.float32(bc * H * W)
    row = sums[0]
    return {
        'main_loss': (row[_LANE_L1_FINE] + row[_LANE_SOB_FINE]) / n,
        'coarse_pred_loss': (row[_LANE_L1_COARSE] + row[_LANE_SOB_COARSE]) / n,
        'pred_err_loss': row[_LANE_ERR] / n,
    }


# speedup vs baseline: 1.0325x; 1.0190x over previous
import functools

import numpy as np

import jax
import jax.numpy as jnp
from jax.experimental import pallas as pl
from jax.experimental.pallas import tpu as pltpu

_SOBEL_EPS = 1e-6

_LANE_L1_FINE = 0
_LANE_SOB_FINE = 1
_LANE_L1_COARSE = 2
_LANE_SOB_COARSE = 3
_LANE_ERR = 4
_OUT_SUBLANES = 8
_OUT_LANES = 128


def _shift_down(x):
    return jnp.concatenate([x[:1], x[:-1]], axis=0)


def _shift_up(x):
    return jnp.concatenate([x[1:], x[-1:]], axis=0)


def _shift_right(x):
    return jnp.concatenate([x[:, :1], x[:, :-1]], axis=1)


def _shift_left(x):
    return jnp.concatenate([x[:, 1:], x[:, -1:]], axis=1)


def _sobel_mag8(x):
    x_u = _shift_down(x)
    x_d = _shift_up(x)
    sv = x_u + 2.0 * x + x_d
    dv = x_d - x_u
    gx = _shift_left(sv) - _shift_right(sv)
    gy = _shift_right(dv) + 2.0 * dv + _shift_left(dv)
    g2 = gx * gx + gy * gy + 64.0 * _SOBEL_EPS
    return g2 * jax.lax.rsqrt(g2)


def _loss_body(p_ref, t_ref, c_ref, e_ref, mh_ref, mw_ref, out_ref,
               *, img_h, low_h, n_img):
    mh = mh_ref[...]
    mw = mw_ref[...]

    def upsample(ref, i):
        xb = ref[i * low_h:(i + 1) * low_h].astype(jnp.bfloat16)
        y = jnp.dot(xb, mw, preferred_element_type=jnp.float32)
        return jnp.dot(mh, y.astype(jnp.bfloat16),
                       preferred_element_type=jnp.float32)

    sums = [jnp.float32(0.0)] * 5

    fsum = lambda x: jnp.sum(x, dtype=jnp.float32)
    for i in range(n_img):
        sl = slice(i * img_h, (i + 1) * img_h)
        p = p_ref[sl].astype(jnp.bfloat16)
        t = t_ref[sl].astype(jnp.bfloat16)
        c = upsample(c_ref, i).astype(jnp.bfloat16)
        e = upsample(e_ref, i).astype(jnp.bfloat16)

        mag_t = _sobel_mag8(t)
        mag_p = _sobel_mag8(p)
        mag_c = _sobel_mag8(c)

        ct = jnp.abs(c - t)
        sums[0] += fsum(jnp.abs(p - t))
        sums[1] += 0.125 * fsum(jnp.abs(mag_p - mag_t))
        sums[2] += fsum(ct)
        sums[3] += 0.125 * fsum(jnp.abs(mag_c - mag_t))
        sums[4] += fsum(jnp.abs(e - ct))

    lane = jax.lax.broadcasted_iota(
        jnp.int32, (_OUT_SUBLANES, _OUT_LANES), 1)
    acc = jnp.zeros((_OUT_SUBLANES, _OUT_LANES), jnp.float32)
    for idx, s in zip((_LANE_L1_FINE, _LANE_SOB_FINE, _LANE_L1_COARSE,
                       _LANE_SOB_COARSE, _LANE_ERR), sums):
        acc = acc + jnp.where(lane == idx, s, 0.0)

    @pl.when(pl.program_id(0) == 0)
    def _init():
        out_ref[...] = acc

    @pl.when(pl.program_id(0) != 0)
    def _accum():
        out_ref[...] += acc


def _bilinear_operator(n_src, n_dst):
    i = np.arange(n_dst, dtype=np.float64)
    src = (i + 0.5) * (n_src / n_dst) - 0.5
    lo = np.floor(src).astype(np.int64)
    w = src - lo
    hi = np.clip(lo + 1, 0, n_src - 1)
    lo = np.clip(lo, 0, n_src - 1)
    m = np.zeros((n_dst, n_src), dtype=np.float32)
    m[i.astype(np.int64), lo] += (1.0 - w).astype(np.float32)
    m[i.astype(np.int64), hi] += w.astype(np.float32)
    return m


def kernel(pred_pha_fine, true_pha, pred_pha_corse, pred_err):
    B, C, H, W = true_pha.shape
    bc = B * C
    hh, ww = pred_pha_corse.shape[2:]
    tbc = next(d for d in (8, 4, 2, 1) if bc % d == 0)
    nblk = bc // tbc

    p2 = pred_pha_fine.reshape(bc * H, W)
    t2 = true_pha.reshape(bc * H, W)
    c2 = pred_pha_corse.reshape(bc * hh, ww)
    e2 = pred_err.reshape(bc * hh, ww)
    mh = jnp.asarray(_bilinear_operator(hh, H), dtype=jnp.bfloat16)
    mw = jnp.asarray(_bilinear_operator(ww, W).T, dtype=jnp.bfloat16)

    full_spec = pl.BlockSpec((tbc * H, W), lambda i: (i, 0))
    half_spec = pl.BlockSpec((tbc * hh, ww), lambda i: (i, 0))
    fixed = lambda shape: pl.BlockSpec(shape, lambda i: (0, 0))

    out_spec = pl.BlockSpec((_OUT_SUBLANES, _OUT_LANES), lambda i: (0, 0))
    out_shape = jax.ShapeDtypeStruct((_OUT_SUBLANES, _OUT_LANES), jnp.float32)

    body = functools.partial(_loss_body, img_h=H, low_h=hh, n_img=tbc)
    sums = pl.pallas_call(
        body,
        out_shape=out_shape,
        grid=(nblk,),
        in_specs=[full_spec, full_spec, half_spec, half_spec,
                  fixed(mh.shape), fixed(mw.shape)],
        out_specs=out_spec,
        compiler_params=pltpu.CompilerParams(
            dimension_semantics=("arbitrary",),
            vmem_limit_bytes=96 * 1024 * 1024),
    )(p2, t2, c2, e2, mh, mw)

    n = jnp.float32(bc * H * W)
    row = sums[0]
    return {
        'main_loss': (row[_LANE_L1_FINE] + row[_LANE_SOB_FINE]) / n,
        'coarse_pred_loss': (row[_LANE_L1_COARSE] + row[_LANE_SOB_COARSE]) / n,
        'pred_err_loss': row[_LANE_ERR] / n,
    }


# speedup vs baseline: 1.0636x; 1.0301x over previous
import functools

import numpy as np

import jax
import jax.numpy as jnp
from jax.experimental import pallas as pl
from jax.experimental.pallas import tpu as pltpu

_SOBEL_EPS = 1e-6

_LANE_L1_FINE = 0
_LANE_SOB_FINE = 1
_LANE_L1_COARSE = 2
_LANE_SOB_COARSE = 3
_LANE_ERR = 4
_OUT_SUBLANES = 8
_OUT_LANES = 128


def _shift_down(x):
    return jnp.concatenate([x[:1], x[:-1]], axis=0)


def _shift_up(x):
    return jnp.concatenate([x[1:], x[-1:]], axis=0)


def _shift_right(x):
    return jnp.concatenate([x[:, :1], x[:, :-1]], axis=1)


def _shift_left(x):
    return jnp.concatenate([x[:, 1:], x[:, -1:]], axis=1)


def _sobel_mag8(x):
    x_u = _shift_down(x)
    x_d = _shift_up(x)
    sv = x_u + 2.0 * x + x_d
    dv = x_d - x_u
    gx = _shift_left(sv) - _shift_right(sv)
    gy = _shift_right(dv) + 2.0 * dv + _shift_left(dv)
    g2 = gx * gx + gy * gy + 64.0 * _SOBEL_EPS
    return g2 * jax.lax.rsqrt(g2)


def _psum(x):
    h = x.shape[0]
    x = x[:h // 2] + x[h // 2:]
    x = x[:, :_OUT_LANES] + x[:, _OUT_LANES:]
    h //= 2
    x = x[:h // 2] + x[h // 2:]
    x = x.astype(jnp.float32)
    h //= 2
    while h > _OUT_SUBLANES:
        x = x[:h // 2] + x[h // 2:]
        h //= 2
    return x


def _loss_body(p_ref, t_ref, c_ref, e_ref, mh_ref, mw_ref, out_ref, acc_ref,
               *, img_h, low_h, n_img, nblk):
    mh = mh_ref[...]
    mw = mw_ref[...]

    def upsample(ref, i):
        xb = ref[i * low_h:(i + 1) * low_h].astype(jnp.bfloat16)
        y = jnp.dot(xb, mw, preferred_element_type=jnp.float32)
        return jnp.dot(mh, y.astype(jnp.bfloat16),
                       preferred_element_type=jnp.float32)

    sums = [jnp.zeros((_OUT_SUBLANES, _OUT_LANES), jnp.float32)] * 5
    for i in range(n_img):
        sl = slice(i * img_h, (i + 1) * img_h)
        p = p_ref[sl].astype(jnp.bfloat16)
        t = t_ref[sl].astype(jnp.bfloat16)
        c = upsample(c_ref, i).astype(jnp.bfloat16)
        e = upsample(e_ref, i).astype(jnp.bfloat16)

        mag_t = _sobel_mag8(t)
        mag_p = _sobel_mag8(p)
        mag_c = _sobel_mag8(c)

        ct = jnp.abs(c - t)
        sums[0] = sums[0] + _psum(jnp.abs(p - t))
        sums[1] = sums[1] + _psum(jnp.abs(mag_p - mag_t))
        sums[2] = sums[2] + _psum(ct)
        sums[3] = sums[3] + _psum(jnp.abs(mag_c - mag_t))
        sums[4] = sums[4] + _psum(jnp.abs(e - ct))

    part = jnp.concatenate(sums, axis=0)
    pid = pl.program_id(0)

    @pl.when(pid == 0)
    def _init():
        acc_ref[...] = part

    @pl.when(pid != 0)
    def _accum():
        acc_ref[...] += part

    @pl.when(pid == nblk - 1)
    def _finalize():
        tot = acc_ref[...]
        lane = jax.lax.broadcasted_iota(
            jnp.int32, (_OUT_SUBLANES, _OUT_LANES), 1)
        out = jnp.zeros((_OUT_SUBLANES, _OUT_LANES), jnp.float32)
        for k, scale in ((_LANE_L1_FINE, 1.0), (_LANE_SOB_FINE, 0.125),
                         (_LANE_L1_COARSE, 1.0), (_LANE_SOB_COARSE, 0.125),
                         (_LANE_ERR, 1.0)):
            s = scale * jnp.sum(
                tot[k * _OUT_SUBLANES:(k + 1) * _OUT_SUBLANES])
            out = out + jnp.where(lane == k, s, 0.0)
        out_ref[...] = out


def _bilinear_operator(n_src, n_dst):
    i = np.arange(n_dst, dtype=np.float64)
    src = (i + 0.5) * (n_src / n_dst) - 0.5
    lo = np.floor(src).astype(np.int64)
    w = src - lo
    hi = np.clip(lo + 1, 0, n_src - 1)
    lo = np.clip(lo, 0, n_src - 1)
    m = np.zeros((n_dst, n_src), dtype=np.float32)
    m[i.astype(np.int64), lo] += (1.0 - w).astype(np.float32)
    m[i.astype(np.int64), hi] += w.astype(np.float32)
    return m


def kernel(pred_pha_fine, true_pha, pred_pha_corse, pred_err):
    B, C, H, W = true_pha.shape
    bc = B * C
    hh, ww = pred_pha_corse.shape[2:]
    tbc = next(d for d in (8, 4, 2, 1) if bc % d == 0)
    nblk = bc // tbc

    p2 = pred_pha_fine.reshape(bc * H, W)
    t2 = true_pha.reshape(bc * H, W)
    c2 = pred_pha_corse.reshape(bc * hh, ww)
    e2 = pred_err.reshape(bc * hh, ww)
    mh = jnp.asarray(_bilinear_operator(hh, H), dtype=jnp.bfloat16)
    mw = jnp.asarray(_bilinear_operator(ww, W).T, dtype=jnp.bfloat16)

    full_spec = pl.BlockSpec((tbc * H, W), lambda i: (i, 0))
    half_spec = pl.BlockSpec((tbc * hh, ww), lambda i: (i, 0))
    fixed = lambda shape: pl.BlockSpec(shape, lambda i: (0, 0))

    out_spec = pl.BlockSpec((_OUT_SUBLANES, _OUT_LANES), lambda i: (0, 0))
    out_shape = jax.ShapeDtypeStruct((_OUT_SUBLANES, _OUT_LANES), jnp.float32)

    body = functools.partial(_loss_body, img_h=H, low_h=hh, n_img=tbc,
                             nblk=nblk)
    sums = pl.pallas_call(
        body,
        out_shape=out_shape,
        grid=(nblk,),
        in_specs=[full_spec, full_spec, half_spec, half_spec,
                  fixed(mh.shape), fixed(mw.shape)],
        out_specs=out_spec,
        scratch_shapes=[pltpu.VMEM((5 * _OUT_SUBLANES, _OUT_LANES),
                                   jnp.float32)],
        compiler_params=pltpu.CompilerParams(
            dimension_semantics=("arbitrary",),
            vmem_limit_bytes=96 * 1024 * 1024),
    )(p2, t2, c2, e2, mh, mw)

    n = jnp.float32(bc * H * W)
    row = sums[0]
    return {
        'main_loss': (row[_LANE_L1_FINE] + row[_LANE_SOB_FINE]) / n,
        'coarse_pred_loss': (row[_LANE_L1_COARSE] + row[_LANE_SOB_COARSE]) / n,
        'pred_err_loss': row[_LANE_ERR] / n,
    }
